# Initial kernel scaffold; baseline (speedup 1.0000x reference)
import jax
import jax.numpy as jnp
from jax.experimental import pallas as pl


def kernel(x, edge_index, W1, b1, W2, b2, W3, b3):
    raise NotImplementedError("write your pallas kernel here")


# speedup vs baseline: 10.2109x; 10.2109x over previous
import jax
import jax.numpy as jnp
from jax import lax
from jax.experimental import pallas as pl
from jax.experimental.pallas import tpu as pltpu
from jax.experimental.pallas import tpu_sc as plsc

N = 10000
E = 160000
D = 256
H = D // 2
NC = 2
NS = 16
R = 1000
NB = N // R
RPW = 624
TAIL = N - NS * RPW
CH = 80
NCHUNK = (E // NS) // CH
CHP = 100
NCHUNKP = (E // (NC * NS)) // CHP

_F32 = jnp.float32


def _p2_body(d0_ref, d1_ref, dis_ref):
    deg = d0_ref[:, :16] + d1_ref[:, :16] + 1.0
    dis_ref[...] = lax.rsqrt(deg)


def _a1_body(x_ref, w_ref, dis_ref, g_ref):
    dis = dis_ref[:, 0:1]
    h = jnp.dot(x_ref[...], w_ref[...], preferred_element_type=_F32) * dis
    g_ref[0] = h[:, :H]
    g_ref[1] = h[:, H:]


def _a23_body(a0_ref, a1_ref, dis_ref, b_ref, w_ref, g_ref):
    dis = dis_ref[:, 0:1]
    acc = jnp.concatenate([a0_ref[...], a1_ref[...]], axis=1)
    xin = jnp.maximum(acc * dis + b_ref[...], 0.0)
    h = jnp.dot(xin, w_ref[...], preferred_element_type=_F32) * dis
    g_ref[0] = h[:, :H]
    g_ref[1] = h[:, H:]


def _c3_body(a0_ref, a1_ref, dis_ref, b_ref, y_ref):
    dis = dis_ref[:, 0:1]
    acc = jnp.concatenate([a0_ref[...], a1_ref[...]], axis=1)
    y_ref[...] = acc * dis + b_ref[...]


_dis_spec = pl.BlockSpec((R, 16), lambda i: (i, 0))
_acc_spec = pl.BlockSpec((R, H), lambda i: (i, 0))
_acc_spec_hi = pl.BlockSpec((R, H), lambda i: (i + NB, 0))
_w_spec = pl.BlockSpec((D, D), lambda i: (0, 0))
_b_spec = pl.BlockSpec((1, D), lambda i: (0, 0))
_g_out_spec = pl.BlockSpec((NC, R, H), lambda i: (0, i, 0))
_g_out_shape = jax.ShapeDtypeStruct((NC, N, H), _F32)

_p2 = pl.pallas_call(
    _p2_body,
    grid=(NB,),
    in_specs=[pl.BlockSpec((R, H), lambda i: (i, 0)),
              pl.BlockSpec((R, H), lambda i: (i + NB, 0))],
    out_specs=pl.BlockSpec((R, 16), lambda i: (i, 0)),
    out_shape=jax.ShapeDtypeStruct((N, 16), _F32),
)

_a1 = pl.pallas_call(
    _a1_body,
    grid=(NB,),
    in_specs=[pl.BlockSpec((R, D), lambda i: (i, 0)), _w_spec, _dis_spec],
    out_specs=_g_out_spec,
    out_shape=_g_out_shape,
)

_a23 = pl.pallas_call(
    _a23_body,
    grid=(NB,),
    in_specs=[_acc_spec, _acc_spec_hi, _dis_spec, _b_spec, _w_spec],
    out_specs=_g_out_spec,
    out_shape=_g_out_shape,
)

_c3 = pl.pallas_call(
    _c3_body,
    grid=(NB,),
    in_specs=[_acc_spec, _acc_spec_hi, _dis_spec, _b_spec],
    out_specs=pl.BlockSpec((R, D), lambda i: (i, 0)),
    out_shape=jax.ShapeDtypeStruct((N, D), _F32),
)


_mesh = plsc.VectorSubcoreMesh(
    core_axis_name="c", subcore_axis_name="s", num_cores=NC, num_subcores=NS)


def _rows_copy(src_ref, dst_ref, s, src_base=0, dst_base=0, add=False):
    r0 = s * RPW
    pltpu.sync_copy(src_ref.at[pl.ds(pl.multiple_of(src_base + r0, 8), RPW)],
                    dst_ref.at[pl.ds(pl.multiple_of(dst_base + r0, 8), RPW)],
                    add=add)

    @pl.when(s == NS - 1)
    def _():
        t0 = NS * RPW
        pltpu.sync_copy(
            src_ref.at[pl.ds(pl.multiple_of(src_base + t0, 8), TAIL)],
            dst_ref.at[pl.ds(pl.multiple_of(dst_base + t0, 8), TAIL)],
            add=add)


def _deg_body(dstp_ref, zeros_ref, ones_ref, deg_ref,
              acc_sh, dst_v, ones_v):
    c = lax.axis_index("c")
    s = lax.axis_index("s")
    w = c * NS + s
    pltpu.sync_copy(dstp_ref.at[w], dst_v)
    pltpu.sync_copy(ones_ref, ones_v)
    _rows_copy(zeros_ref, acc_sh, s)
    plsc.subcore_barrier()

    def chunk(j, carry):
        pltpu.sync_copy(ones_v, acc_sh.at[dst_v.at[j]], add=True)
        return carry

    lax.fori_loop(0, NCHUNKP, chunk, 0)
    plsc.subcore_barrier()
    _rows_copy(acc_sh, deg_ref, s, dst_base=c * N)


_deg_call = pl.kernel(
    _deg_body,
    out_type=jax.ShapeDtypeStruct((NC * N, H), _F32),
    mesh=_mesh,
    scratch_types=[
        pltpu.VMEM_SHARED((N, H), _F32),
        pltpu.VMEM((NCHUNKP, CHP), jnp.int32),
        pltpu.VMEM((CHP, H), _F32),
    ],
)


def _prop_body(g_ref, srcp_ref, dstp_ref, acc_ref,
               acc_sh, src_v, dst_v, buf, sem):
    c = lax.axis_index("c")
    s = lax.axis_index("s")
    pltpu.sync_copy(srcp_ref.at[c * NS + s], src_v)
    pltpu.sync_copy(dstp_ref.at[s], dst_v)
    base = c * N
    _rows_copy(g_ref, acc_sh, s, src_base=base)
    plsc.subcore_barrier()

    def chunk(j, carry):
        pltpu.async_copy(g_ref.at[src_v.at[j]], buf, sem).wait()
        pltpu.sync_copy(buf, acc_sh.at[dst_v.at[j]], add=True)
        return carry

    lax.fori_loop(0, NCHUNK, chunk, 0)
    plsc.subcore_barrier()
    _rows_copy(acc_sh, acc_ref, s, dst_base=base)


_prop = pl.kernel(
    _prop_body,
    out_type=jax.ShapeDtypeStruct((NC * N, H), _F32),
    mesh=_mesh,
    scratch_types=[
        pltpu.VMEM_SHARED((N, H), _F32),
        pltpu.VMEM((NCHUNK, CH), jnp.int32),
        pltpu.VMEM((NCHUNK, CH), jnp.int32),
        pltpu.VMEM((CH, H), _F32),
        pltpu.SemaphoreType.DMA,
    ],
)


def kernel(x, edge_index, W1, b1, W2, b2, W3, b3):
    src = edge_index[0]
    dst = edge_index[1]
    dstp_deg = dst.reshape(NC * NS, NCHUNKP, CHP)
    src_lo = src.reshape(NS, NCHUNK, CH)
    srcp = jnp.concatenate([src_lo, src_lo + N], axis=0)
    dstp = dst.reshape(NS, NCHUNK, CH)
    zerosw = jnp.zeros((N, H), _F32)
    onesw = jnp.ones((CHP, H), _F32)

    degw = _deg_call(dstp_deg, zerosw, onesw)
    dis = _p2(degw, degw)
    g = _a1(x, W1, dis).reshape(NC * N, H)
    acc = _prop(g, srcp, dstp)
    g = _a23(acc, acc, dis, b1.reshape(1, D), W2).reshape(NC * N, H)
    acc = _prop(g, srcp, dstp)
    g = _a23(acc, acc, dis, b2.reshape(1, D), W3).reshape(NC * N, H)
    acc = _prop(g, srcp, dstp)
    return _c3(acc, acc, dis, b3.reshape(1, D))


# speedup vs baseline: 14.7507x; 1.4446x over previous
import jax
import jax.numpy as jnp
from jax import lax
from jax.experimental import pallas as pl
from jax.experimental.pallas import tpu as pltpu
from jax.experimental.pallas import tpu_sc as plsc

N = 10000
E = 160000
D = 256
H = D // 2
NC = 2
NS = 16
R = 1000
NB = N // R
RPW = 624
TAIL = N - NS * RPW
CH = 125
NCHUNK = (E // NS) // CH
CHP = 100
NCHUNKP = (E // (NC * NS)) // CHP

_F32 = jnp.float32


def _p2_body(d0_ref, d1_ref, dis_ref):
    deg = d0_ref[:, :16] + d1_ref[:, :16] + 1.0
    dis_ref[...] = lax.rsqrt(deg)


def _a1_body(x_ref, w_ref, dis_ref, g_ref):
    dis = dis_ref[:, 0:1]
    h = jnp.dot(x_ref[...], w_ref[...], preferred_element_type=_F32) * dis
    g_ref[0] = h[:, :H]
    g_ref[1] = h[:, H:]


def _a23_body(a0_ref, a1_ref, dis_ref, b_ref, w_ref, g_ref):
    dis = dis_ref[:, 0:1]
    acc = jnp.concatenate([a0_ref[...], a1_ref[...]], axis=1)
    xin = jnp.maximum(acc * dis + b_ref[...], 0.0)
    h = jnp.dot(xin, w_ref[...], preferred_element_type=_F32) * dis
    g_ref[0] = h[:, :H]
    g_ref[1] = h[:, H:]


def _c3_body(a0_ref, a1_ref, dis_ref, b_ref, y_ref):
    dis = dis_ref[:, 0:1]
    acc = jnp.concatenate([a0_ref[...], a1_ref[...]], axis=1)
    y_ref[...] = acc * dis + b_ref[...]


_dis_spec = pl.BlockSpec((R, 16), lambda i: (i, 0))
_acc_spec = pl.BlockSpec((R, H), lambda i: (i, 0))
_acc_spec_hi = pl.BlockSpec((R, H), lambda i: (i + NB, 0))
_w_spec = pl.BlockSpec((D, D), lambda i: (0, 0))
_b_spec = pl.BlockSpec((1, D), lambda i: (0, 0))
_g_out_spec = pl.BlockSpec((NC, R, H), lambda i: (0, i, 0))
_g_out_shape = jax.ShapeDtypeStruct((NC, N, H), _F32)

_p2 = pl.pallas_call(
    _p2_body,
    grid=(NB,),
    in_specs=[pl.BlockSpec((R, H), lambda i: (i, 0)),
              pl.BlockSpec((R, H), lambda i: (i + NB, 0))],
    out_specs=pl.BlockSpec((R, 16), lambda i: (i, 0)),
    out_shape=jax.ShapeDtypeStruct((N, 16), _F32),
)

_a1 = pl.pallas_call(
    _a1_body,
    grid=(NB,),
    in_specs=[pl.BlockSpec((R, D), lambda i: (i, 0)), _w_spec, _dis_spec],
    out_specs=_g_out_spec,
    out_shape=_g_out_shape,
)

_a23 = pl.pallas_call(
    _a23_body,
    grid=(NB,),
    in_specs=[_acc_spec, _acc_spec_hi, _dis_spec, _b_spec, _w_spec],
    out_specs=_g_out_spec,
    out_shape=_g_out_shape,
)

_c3 = pl.pallas_call(
    _c3_body,
    grid=(NB,),
    in_specs=[_acc_spec, _acc_spec_hi, _dis_spec, _b_spec],
    out_specs=pl.BlockSpec((R, D), lambda i: (i, 0)),
    out_shape=jax.ShapeDtypeStruct((N, D), _F32),
)


_mesh = plsc.VectorSubcoreMesh(
    core_axis_name="c", subcore_axis_name="s", num_cores=NC, num_subcores=NS)


def _rows_copy(src_ref, dst_ref, s, src_base=0, dst_base=0, add=False):
    r0 = s * RPW
    pltpu.sync_copy(src_ref.at[pl.ds(pl.multiple_of(src_base + r0, 8), RPW)],
                    dst_ref.at[pl.ds(pl.multiple_of(dst_base + r0, 8), RPW)],
                    add=add)

    @pl.when(s == NS - 1)
    def _():
        t0 = NS * RPW
        pltpu.sync_copy(
            src_ref.at[pl.ds(pl.multiple_of(src_base + t0, 8), TAIL)],
            dst_ref.at[pl.ds(pl.multiple_of(dst_base + t0, 8), TAIL)],
            add=add)


def _deg_body(dstp_ref, zeros_ref, ones_ref, deg_ref,
              acc_sh, dst_v, ones_v):
    c = lax.axis_index("c")
    s = lax.axis_index("s")
    w = c * NS + s
    pltpu.sync_copy(dstp_ref.at[w], dst_v)
    pltpu.sync_copy(ones_ref, ones_v)
    _rows_copy(zeros_ref, acc_sh, s)
    plsc.subcore_barrier()

    def chunk(j, carry):
        pltpu.sync_copy(ones_v, acc_sh.at[dst_v.at[j]], add=True)
        return carry

    lax.fori_loop(0, NCHUNKP, chunk, 0)
    plsc.subcore_barrier()
    _rows_copy(acc_sh, deg_ref, s, dst_base=c * N)


_deg_call = pl.kernel(
    _deg_body,
    out_type=jax.ShapeDtypeStruct((NC * N, H), _F32),
    mesh=_mesh,
    scratch_types=[
        pltpu.VMEM_SHARED((N, H), _F32),
        pltpu.VMEM((NCHUNKP, CHP), jnp.int32),
        pltpu.VMEM((CHP, H), _F32),
    ],
)


def _prop_body(g_ref, srcw_ref, dstw_ref, acc_ref,
               acc_sh, src_w, dst_w, buf, sem_g, sem_i):
    c = lax.axis_index("c")
    s = lax.axis_index("s")
    sb = (c * NS + s) * NCHUNK
    db = s * NCHUNK
    base = c * N
    _rows_copy(g_ref, acc_sh, s, src_base=base)
    pltpu.sync_copy(srcw_ref.at[sb], src_w.at[0])
    pltpu.sync_copy(dstw_ref.at[db], dst_w.at[0])
    plsc.subcore_barrier()
    pltpu.async_copy(g_ref.at[src_w.at[0, 0]], buf.at[0], sem_g)
    pltpu.async_copy(srcw_ref.at[sb + 1], src_w.at[1], sem_i)
    pltpu.async_copy(dstw_ref.at[db + 1], dst_w.at[1], sem_i)

    def chunk(j, carry):
        jm = lax.rem(j, 2)
        jn = 1 - jm
        pltpu.make_async_copy(g_ref.at[src_w.at[jm, 0]], buf.at[jm],
                              sem_g).wait()

        @pl.when(j + 1 < NCHUNK)
        def _():
            pltpu.make_async_copy(srcw_ref.at[sb + j + 1], src_w.at[jn],
                                  sem_i).wait()
            pltpu.make_async_copy(dstw_ref.at[db + j + 1], dst_w.at[jn],
                                  sem_i).wait()
            pltpu.async_copy(g_ref.at[src_w.at[jn, 0]], buf.at[jn], sem_g)

        pltpu.sync_copy(buf.at[jm], acc_sh.at[dst_w.at[jm, 0]], add=True)

        @pl.when(j + 2 < NCHUNK)
        def _():
            pltpu.async_copy(srcw_ref.at[sb + j + 2], src_w.at[jm], sem_i)
            pltpu.async_copy(dstw_ref.at[db + j + 2], dst_w.at[jm], sem_i)

        return carry

    lax.fori_loop(0, NCHUNK, chunk, 0)
    plsc.subcore_barrier()
    _rows_copy(acc_sh, acc_ref, s, dst_base=base)


_prop = pl.kernel(
    _prop_body,
    out_type=jax.ShapeDtypeStruct((NC * N, H), _F32),
    mesh=_mesh,
    scratch_types=[
        pltpu.VMEM_SHARED((N, H), _F32),
        pltpu.VMEM((2, 1, CH), jnp.int32),
        pltpu.VMEM((2, 1, CH), jnp.int32),
        pltpu.VMEM((2, CH, H), _F32),
        pltpu.SemaphoreType.DMA,
        pltpu.SemaphoreType.DMA,
    ],
)


def kernel(x, edge_index, W1, b1, W2, b2, W3, b3):
    src = edge_index[0]
    dst = edge_index[1]
    dstp_deg = dst.reshape(NC * NS, NCHUNKP, CHP)
    src_lo = src.reshape(NS, NCHUNK, CH)
    srcw = jnp.concatenate([src_lo, src_lo + N],
                           axis=0).reshape(NC * NS * NCHUNK, 1, CH)
    dstw = dst.reshape(NS * NCHUNK, 1, CH)
    zerosw = jnp.zeros((N, H), _F32)
    onesw = jnp.ones((CHP, H), _F32)

    degw = _deg_call(dstp_deg, zerosw, onesw)
    dis = _p2(degw, degw)
    g = _a1(x, W1, dis).reshape(NC * N, H)
    acc = _prop(g, srcw, dstw)
    g = _a23(acc, acc, dis, b1.reshape(1, D), W2).reshape(NC * N, H)
    acc = _prop(g, srcw, dstw)
    g = _a23(acc, acc, dis, b2.reshape(1, D), W3).reshape(NC * N, H)
    acc = _prop(g, srcw, dstw)
    return _c3(acc, acc, dis, b3.reshape(1, D))
